# baseline (speedup 1.0000x reference)
import functools

import jax
import jax.numpy as jnp
from jax import lax
from jax.experimental import pallas as pl
from jax.experimental.pallas import tpu as pltpu
from jax.experimental.pallas import tpu_sc as plsc

N = 10000
E = 320000
D_IN = 128
D_E = 16
N_CLASSES = 4
H1, H2, H3 = 5, 10, 10

NC = 2
NS = 16
NW = NC * NS
C = 1280
NCHUNK = E // C
RPT = 624

_mesh = plsc.VectorSubcoreMesh(core_axis_name="c", subcore_axis_name="s")
_sc_params = pltpu.CompilerParams(needs_layout_passes=False,
                                  use_tc_tiling_on_sc=False)


def _iota16():
    return lax.iota(jnp.int32, 16)


def _full16(v):
    return jnp.full((16,), v, jnp.int32)


def _nk(wid):
    q, r = NCHUNK // NW, NCHUNK % NW
    if r == 0:
        return q
    return jnp.where(wid < r, q + 1, q)


def _make_sc_gat(ST, H):
    CG = 400
    NKG = 25

    @functools.partial(
        pl.kernel,
        out_type=jax.ShapeDtypeStruct((NC, N, ST), jnp.float32),
        mesh=_mesh,
        compiler_params=_sc_params,
        scratch_types=[
            pltpu.VMEM((N,), jnp.float32),
            pltpu.VMEM((CG,), jnp.int32),
            pltpu.VMEM((CG,), jnp.int32),
            pltpu.VMEM((CG, ST), jnp.float32),
            pltpu.VMEM((CG,), jnp.int32),
            pltpu.VMEM((CG,), jnp.int32),
            pltpu.VMEM((CG, ST), jnp.float32),
            pltpu.VMEM_SHARED((N, ST), jnp.float32),
            pltpu.SemaphoreType.DMA,
            pltpu.SemaphoreType.DMA,
        ],
    )
    def gat(table_hbm, d_hbm, src_hbm, dst_hbm, out_hbm,
            d_v, src_v0, dst_v0, rows_v0, src_v1, dst_v1, rows_v1,
            acc_sh, sem0, sem1):
        cid = lax.axis_index("c")
        sid = lax.axis_index("s")
        wid = sid * NC + cid
        ebase = wid * (NKG * CG)
        bufs = ((src_v0, dst_v0, rows_v0, sem0),
                (src_v1, dst_v1, rows_v1, sem1))

        z = jnp.zeros((16,), jnp.float32)
        ziota = _iota16()

        def zrow(t, carry):
            p = t * 16 + ziota
            plsc.store_scatter(rows_v0, [p // ST, p % ST], z)
            return carry

        lax.fori_loop(0, CG * ST // 16, zrow, 0)
        a0 = sid * RPT
        pltpu.sync_copy(rows_v0.at[pl.ds(0, CG)], acc_sh.at[pl.ds(a0, CG)])
        pltpu.sync_copy(rows_v0.at[pl.ds(0, RPT - CG)],
                        acc_sh.at[pl.ds(a0 + CG, RPT - CG)])

        @pl.when(sid == NS - 1)
        def _ztail():
            pltpu.sync_copy(rows_v0.at[pl.ds(0, 16)],
                            acc_sh.at[pl.ds(N - 16, 16)])

        pltpu.sync_copy(d_hbm, d_v)
        plsc.subcore_barrier()

        iota = _iota16()

        def start_gather(k, b):
            src_v, dst_v, rows_v, sem = bufs[b]
            base = ebase + k * CG
            pltpu.sync_copy(src_hbm.at[pl.ds(base, CG)], src_v)
            pltpu.sync_copy(dst_hbm.at[pl.ds(base, CG)], dst_v)
            pltpu.async_copy(table_hbm.at[src_v], rows_v, sem)

        def finish_chunk(b):
            src_v, dst_v, rows_v, sem = bufs[b]
            pltpu.make_async_copy(table_hbm.at[src_v], rows_v, sem).wait()
            for i in range(CG // 16):
                off = i * 16
                jrow = iota + off
                dj = dst_v[pl.ds(off, 16)]
                sv = plsc.load_gather(rows_v, [jrow, _full16(H)])
                dv = plsc.load_gather(d_v, [dj])
                l = sv + dv
                l = jnp.where(l >= 0.0, l, 0.2 * l)
                w = jnp.exp(l)
                for f in range(H):
                    cf = plsc.load_gather(rows_v, [jrow, _full16(f)])
                    plsc.store_scatter(rows_v, [jrow, _full16(f)], cf * w)
                plsc.store_scatter(rows_v, [jrow, _full16(H)], w)
                plsc.store_scatter(rows_v, [jrow, _full16(H + 1)], w)
            pltpu.sync_copy(rows_v, acc_sh.at[dst_v], add=True)

        start_gather(0, 0)

        def pair_body(k2, carry):
            k = 2 * k2
            start_gather(k + 1, 1)
            finish_chunk(0)
            start_gather(k + 2, 0)
            finish_chunk(1)
            return carry

        lax.fori_loop(0, (NKG - 1) // 2, pair_body, 0)
        finish_chunk(0)

        plsc.subcore_barrier()
        pltpu.sync_copy(acc_sh.at[pl.ds(a0, RPT)],
                        out_hbm.at[cid, pl.ds(a0, RPT)])

        @pl.when(sid == NS - 1)
        def _tail():
            pltpu.sync_copy(acc_sh.at[pl.ds(N - 16, 16)],
                            out_hbm.at[cid, pl.ds(N - 16, 16)])

    return gat


def _make_sc_edge_mlp():
    @functools.partial(
        pl.kernel,
        out_type=jax.ShapeDtypeStruct((E // 32, 128), jnp.float32),
        mesh=_mesh,
        compiler_params=_sc_params,
        scratch_types=[
            pltpu.VMEM((48,), jnp.float32),
            pltpu.VMEM((C,), jnp.int32),
            pltpu.VMEM((C,), jnp.int32),
            pltpu.VMEM((C, 12), jnp.float32),
            pltpu.VMEM((C, 12), jnp.float32),
            pltpu.VMEM((16, C), jnp.float32),
            pltpu.VMEM((C // 32, 128), jnp.float32),
            pltpu.SemaphoreType.DMA,
            pltpu.SemaphoreType.DMA,
        ],
    )
    def edge_mlp(tabp_hbm, tabq_hbm, rt_hbm, wl_hbm, src_hbm, dst_hbm,
                 out_hbm, wl_v, src_v, dst_v, rp_v, rq_v, rt_v, ob_v,
                 semp, semq):
        cid = lax.axis_index("c")
        sid = lax.axis_index("s")
        wid = sid * NC + cid

        pltpu.sync_copy(wl_hbm, wl_v)
        iota = _iota16()
        w9b = [[plsc.load_gather(wl_v, [_full16(j * 4 + c)])
                for c in range(N_CLASSES)] for j in range(10)]
        b9b = [plsc.load_gather(wl_v, [_full16(40 + c)])
               for c in range(N_CLASSES)]

        def chunk_body(k, carry):
            base = (wid + NW * k) * C
            pltpu.sync_copy(src_hbm.at[pl.ds(base, C)], src_v)
            pltpu.sync_copy(dst_hbm.at[pl.ds(base, C)], dst_v)
            cp = pltpu.async_copy(tabp_hbm.at[src_v], rp_v, semp)
            cq = pltpu.async_copy(tabq_hbm.at[dst_v], rq_v, semq)
            pltpu.sync_copy(rt_hbm.at[:, pl.ds(base, C)], rt_v)
            cp.wait()
            cq.wait()

            def group_body(i, gc):
                off = i * 16
                jrow = iota + off
                e2 = []
                for j in range(10):
                    a = (plsc.load_gather(rp_v, [jrow, _full16(j)])
                         + plsc.load_gather(rq_v, [jrow, _full16(j)])
                         + rt_v[j, pl.ds(off, 16)])
                    e2.append(jnp.maximum(a, 0.0))
                oc = []
                for c in range(N_CLASSES):
                    o = b9b[c]
                    for j in range(10):
                        o = o + e2[j] * w9b[j][c]
                    oc.append(jnp.maximum(o, 0.0))
                m = jnp.maximum(jnp.maximum(oc[0], oc[1]),
                                jnp.maximum(oc[2], oc[3]))
                ex = [jnp.exp(o - m) for o in oc]
                s = ex[0] + ex[1] + ex[2] + ex[3]
                obr = lax.shift_right_logical(jrow, 5)
                obc = lax.shift_left((jrow & 31), 2)
                for c in range(N_CLASSES):
                    plsc.store_scatter(ob_v, [obr, obc + c], ex[c] / s)
                return gc

            lax.fori_loop(0, C // 16, group_body, 0)
            pltpu.sync_copy(ob_v, out_hbm.at[pl.ds(base >> 5, C >> 5)])
            return carry

        lax.fori_loop(0, _nk(wid), chunk_body, 0)

    return edge_mlp


def _tc_prep1_body(x_ref, w_ref, as_ref, ad_ref, o_ref, d_ref):
    W = w_ref[...]
    s_col = W @ as_ref[...]
    d_col = W @ ad_ref[...]
    wbig = jnp.concatenate(
        [W, s_col[:, None], d_col[:, None], jnp.zeros((D_IN, 1), jnp.float32)],
        axis=1)
    t = jnp.dot(x_ref[...], wbig, preferred_element_type=jnp.float32)
    o_ref[...] = t
    d_ref[...] = t[:, H1 + 1]


def tc_prep1(x, W1, a1s, a1d):
    return pl.pallas_call(
        _tc_prep1_body,
        out_shape=(jax.ShapeDtypeStruct((N, 8), jnp.float32),
                   jax.ShapeDtypeStruct((N,), jnp.float32)),
    )(x, W1, a1s, a1d)


def _make_tc_prep_mid(Hp, H, ST):
    def body(acc_ref, b_ref, w_ref, as_ref, ad_ref, o_ref, d_ref):
        a = acc_ref[0] + acc_ref[1]
        den = a[:, Hp:Hp + 1] + 1e-16
        xl = jax.nn.relu(a[:, :Hp] / den + b_ref[...])
        W = w_ref[...]
        s_col = W @ as_ref[...]
        d_col = W @ ad_ref[...]
        pad = ST - (H + 2)
        cols = [W, s_col[:, None], d_col[:, None]]
        if pad:
            cols.append(jnp.zeros((Hp, pad), jnp.float32))
        wbig = jnp.concatenate(cols, axis=1)
        t = jnp.dot(xl, wbig, preferred_element_type=jnp.float32)
        o_ref[...] = t
        d_ref[...] = t[:, H + 1]

    def run(acc, b, W, a_s, a_d):
        return pl.pallas_call(
            body,
            out_shape=(jax.ShapeDtypeStruct((N, ST), jnp.float32),
                       jax.ShapeDtypeStruct((N,), jnp.float32)),
        )(acc, b, W, a_s, a_d)

    return run


def _tc_prep4_body(acc_ref, b_ref, we_ref, p_ref, q_ref):
    a = acc_ref[0] + acc_ref[1]
    den = a[:, H3:H3 + 1] + 1e-16
    x3 = jax.nn.relu(a[:, :H3] / den + b_ref[...])
    zpad = jnp.zeros((N, 2), jnp.float32)
    p = jnp.dot(x3, we_ref[:H3, :], preferred_element_type=jnp.float32)
    q = jnp.dot(x3, we_ref[H3:2 * H3, :], preferred_element_type=jnp.float32)
    p_ref[...] = jnp.concatenate([p, zpad], axis=1)
    q_ref[...] = jnp.concatenate([q, zpad], axis=1)


def tc_prep4(acc, b3, We):
    return pl.pallas_call(
        _tc_prep4_body,
        out_shape=(jax.ShapeDtypeStruct((N, 12), jnp.float32),
                   jax.ShapeDtypeStruct((N, 12), jnp.float32)),
    )(acc, b3, We)


BEP = 16000


def _tc_eproj_body(e_ref, we_ref, be_ref, o_ref):
    r = lax.dot_general(we_ref[...], e_ref[...], (((0,), (1,)), ((), ())),
                        preferred_element_type=jnp.float32)
    r = r + be_ref[...][:, None]
    o_ref[...] = jnp.concatenate(
        [r, jnp.zeros((6, r.shape[1]), jnp.float32)], axis=0)


def tc_eproj(e, We, be):
    return pl.pallas_call(
        _tc_eproj_body,
        grid=(E // BEP,),
        in_specs=[
            pl.BlockSpec((BEP, D_E), lambda i: (i, 0)),
            pl.BlockSpec((D_E, 10), lambda i: (0, 0)),
            pl.BlockSpec((10,), lambda i: (0,)),
        ],
        out_specs=pl.BlockSpec((16, BEP), lambda i: (0, i)),
        out_shape=jax.ShapeDtypeStruct((16, E), jnp.float32),
    )(e, We[2 * H3:], be)


_sc_gat1 = _make_sc_gat(8, H1)
_sc_gat23 = _make_sc_gat(12, H2)
_sc_mlp = _make_sc_edge_mlp()
_tc_prep2 = _make_tc_prep_mid(H1, H2, 12)
_tc_prep3 = _make_tc_prep_mid(H2, H3, 12)


def kernel(x, e, edge_index, W1, a1s, a1d, b1, W2, a2s, a2d, b2,
           W3, a3s, a3d, b3, We, be, W9, b9):
    src = edge_index[0]
    dst = edge_index[1]
    wl = jnp.concatenate(
        [W9.reshape(-1), b9, jnp.zeros((4,), jnp.float32)])

    rt = tc_eproj(e, We, be)
    t1, d1 = tc_prep1(x, W1, a1s, a1d)
    acc1 = _sc_gat1(t1, d1, src, dst)
    t2, d2 = _tc_prep2(acc1, b1, W2, a2s, a2d)
    acc2 = _sc_gat23(t2, d2, src, dst)
    t3, d3 = _tc_prep3(acc2, b2, W3, a3s, a3d)
    acc3 = _sc_gat23(t3, d3, src, dst)
    tp, tq = tc_prep4(acc3, b3, We)
    out2 = _sc_mlp(tp, tq, rt, wl, src, dst)
    return out2.reshape(E, N_CLASSES)


# speedup vs baseline: 1.0871x; 1.0871x over previous
import functools

import jax
import jax.numpy as jnp
from jax import lax
from jax.experimental import pallas as pl
from jax.experimental.pallas import tpu as pltpu
from jax.experimental.pallas import tpu_sc as plsc

N = 10000
E = 320000
D_IN = 128
D_E = 16
N_CLASSES = 4
H1, H2, H3 = 5, 10, 10

NC = 2
NS = 16
NW = NC * NS
C = 1280
NCHUNK = E // C
RPT = 624

_mesh = plsc.VectorSubcoreMesh(core_axis_name="c", subcore_axis_name="s")
_sc_params = pltpu.CompilerParams(needs_layout_passes=False,
                                  use_tc_tiling_on_sc=False)


def _iota16():
    return lax.iota(jnp.int32, 16)


def _full16(v):
    return jnp.full((16,), v, jnp.int32)


def _nk(wid):
    q, r = NCHUNK // NW, NCHUNK % NW
    if r == 0:
        return q
    return jnp.where(wid < r, q + 1, q)


def _make_sc_gat(ST, H):
    CG = 400
    NKG = 25

    @functools.partial(
        pl.kernel,
        out_type=jax.ShapeDtypeStruct((NC, N, ST), jnp.float32),
        mesh=_mesh,
        compiler_params=_sc_params,
        scratch_types=[
            pltpu.VMEM((N,), jnp.float32),
            pltpu.VMEM((CG,), jnp.int32),
            pltpu.VMEM((CG,), jnp.int32),
            pltpu.VMEM((CG, ST), jnp.float32),
            pltpu.VMEM((CG,), jnp.int32),
            pltpu.VMEM((CG,), jnp.int32),
            pltpu.VMEM((CG, ST), jnp.float32),
            pltpu.VMEM_SHARED((N, ST), jnp.float32),
            pltpu.SemaphoreType.DMA,
            pltpu.SemaphoreType.DMA,
        ],
    )
    def gat(table_hbm, d_hbm, src_hbm, dst_hbm, out_hbm,
            d_v, src_v0, dst_v0, rows_v0, src_v1, dst_v1, rows_v1,
            acc_sh, sem0, sem1):
        cid = lax.axis_index("c")
        sid = lax.axis_index("s")
        wid = sid * NC + cid
        ebase = wid * (NKG * CG)
        bufs = ((src_v0, dst_v0, rows_v0, sem0),
                (src_v1, dst_v1, rows_v1, sem1))

        z = jnp.zeros((16,), jnp.float32)
        ziota = _iota16()

        def zrow(t, carry):
            p = t * 16 + ziota
            plsc.store_scatter(rows_v0, [p // ST, p % ST], z)
            return carry

        lax.fori_loop(0, CG * ST // 16, zrow, 0)
        a0 = sid * RPT
        pltpu.sync_copy(rows_v0.at[pl.ds(0, CG)], acc_sh.at[pl.ds(a0, CG)])
        pltpu.sync_copy(rows_v0.at[pl.ds(0, RPT - CG)],
                        acc_sh.at[pl.ds(a0 + CG, RPT - CG)])

        @pl.when(sid == NS - 1)
        def _ztail():
            pltpu.sync_copy(rows_v0.at[pl.ds(0, 16)],
                            acc_sh.at[pl.ds(N - 16, 16)])

        pltpu.sync_copy(d_hbm, d_v)
        plsc.subcore_barrier()

        iota = _iota16()

        def start_gather(k, b):
            src_v, dst_v, rows_v, sem = bufs[b]
            base = ebase + k * CG
            pltpu.sync_copy(src_hbm.at[pl.ds(base, CG)], src_v)
            pltpu.sync_copy(dst_hbm.at[pl.ds(base, CG)], dst_v)
            pltpu.async_copy(table_hbm.at[src_v], rows_v, sem)

        def finish_chunk(b):
            src_v, dst_v, rows_v, sem = bufs[b]
            pltpu.make_async_copy(table_hbm.at[src_v], rows_v, sem).wait()
            for i in range(CG // 16):
                off = i * 16
                jrow = iota + off
                dj = dst_v[pl.ds(off, 16)]
                sv = plsc.load_gather(rows_v, [jrow, _full16(H)])
                dv = plsc.load_gather(d_v, [dj])
                l = sv + dv
                l = jnp.where(l >= 0.0, l, 0.2 * l)
                w = jnp.exp(l)
                for f in range(H):
                    cf = plsc.load_gather(rows_v, [jrow, _full16(f)])
                    plsc.store_scatter(rows_v, [jrow, _full16(f)], cf * w)
                plsc.store_scatter(rows_v, [jrow, _full16(H)], w)
                plsc.store_scatter(rows_v, [jrow, _full16(H + 1)], w)
            pltpu.sync_copy(rows_v, acc_sh.at[dst_v], add=True)

        start_gather(0, 0)

        def pair_body(k2, carry):
            k = 2 * k2
            start_gather(k + 1, 1)
            finish_chunk(0)
            start_gather(k + 2, 0)
            finish_chunk(1)
            return carry

        lax.fori_loop(0, (NKG - 1) // 2, pair_body, 0)
        finish_chunk(0)

        plsc.subcore_barrier()
        pltpu.sync_copy(acc_sh.at[pl.ds(a0, RPT)],
                        out_hbm.at[cid, pl.ds(a0, RPT)])

        @pl.when(sid == NS - 1)
        def _tail():
            pltpu.sync_copy(acc_sh.at[pl.ds(N - 16, 16)],
                            out_hbm.at[cid, pl.ds(N - 16, 16)])

    return gat


def _make_sc_edge_mlp():
    @functools.partial(
        pl.kernel,
        out_type=jax.ShapeDtypeStruct((E, N_CLASSES), jnp.float32),
        mesh=_mesh,
        compiler_params=_sc_params,
        scratch_types=[
            pltpu.VMEM((48,), jnp.float32),
            pltpu.VMEM((C,), jnp.int32),
            pltpu.VMEM((C,), jnp.int32),
            pltpu.VMEM((C, 12), jnp.float32),
            pltpu.VMEM((C, 12), jnp.float32),
            pltpu.VMEM((16, C), jnp.float32),
            pltpu.VMEM((C, N_CLASSES), jnp.float32),
            pltpu.SemaphoreType.DMA,
            pltpu.SemaphoreType.DMA,
        ],
    )
    def edge_mlp(tabp_hbm, tabq_hbm, rt_hbm, wl_hbm, src_hbm, dst_hbm,
                 out_hbm, wl_v, src_v, dst_v, rp_v, rq_v, rt_v, ob_v,
                 semp, semq):
        cid = lax.axis_index("c")
        sid = lax.axis_index("s")
        wid = sid * NC + cid

        pltpu.sync_copy(wl_hbm, wl_v)
        iota = _iota16()
        w9b = [[plsc.load_gather(wl_v, [_full16(j * 4 + c)])
                for c in range(N_CLASSES)] for j in range(10)]
        b9b = [plsc.load_gather(wl_v, [_full16(40 + c)])
               for c in range(N_CLASSES)]

        def chunk_body(k, carry):
            base = (wid + NW * k) * C
            pltpu.sync_copy(src_hbm.at[pl.ds(base, C)], src_v)
            pltpu.sync_copy(dst_hbm.at[pl.ds(base, C)], dst_v)
            cp = pltpu.async_copy(tabp_hbm.at[src_v], rp_v, semp)
            cq = pltpu.async_copy(tabq_hbm.at[dst_v], rq_v, semq)
            pltpu.sync_copy(rt_hbm.at[:, pl.ds(base, C)], rt_v)
            cp.wait()
            cq.wait()

            def group_body(i, gc):
                off = i * 16
                jrow = iota + off
                e2 = []
                for j in range(10):
                    a = (plsc.load_gather(rp_v, [jrow, _full16(j)])
                         + plsc.load_gather(rq_v, [jrow, _full16(j)])
                         + rt_v[j, pl.ds(off, 16)])
                    e2.append(jnp.maximum(a, 0.0))
                oc = []
                for c in range(N_CLASSES):
                    o = b9b[c]
                    for j in range(10):
                        o = o + e2[j] * w9b[j][c]
                    oc.append(jnp.maximum(o, 0.0))
                m = jnp.maximum(jnp.maximum(oc[0], oc[1]),
                                jnp.maximum(oc[2], oc[3]))
                ex = [jnp.exp(o - m) for o in oc]
                s = ex[0] + ex[1] + ex[2] + ex[3]
                for c in range(N_CLASSES):
                    plsc.store_scatter(ob_v, [jrow, _full16(c)], ex[c] / s)
                return gc

            lax.fori_loop(0, C // 16, group_body, 0)
            pltpu.sync_copy(ob_v, out_hbm.at[pl.ds(base, C)])
            return carry

        lax.fori_loop(0, _nk(wid), chunk_body, 0)

    return edge_mlp


def _tc_prep1_body(x_ref, w_ref, as_ref, ad_ref, o_ref, d_ref):
    W = w_ref[...]
    s_col = W @ as_ref[...]
    d_col = W @ ad_ref[...]
    wbig = jnp.concatenate(
        [W, s_col[:, None], d_col[:, None], jnp.zeros((D_IN, 1), jnp.float32)],
        axis=1)
    t = jnp.dot(x_ref[...], wbig, preferred_element_type=jnp.float32)
    o_ref[...] = t
    d_ref[...] = t[:, H1 + 1]


def tc_prep1(x, W1, a1s, a1d):
    return pl.pallas_call(
        _tc_prep1_body,
        out_shape=(jax.ShapeDtypeStruct((N, 8), jnp.float32),
                   jax.ShapeDtypeStruct((N,), jnp.float32)),
    )(x, W1, a1s, a1d)


def _make_tc_prep_mid(Hp, H, ST):
    def body(acc_ref, b_ref, w_ref, as_ref, ad_ref, o_ref, d_ref):
        a = acc_ref[0] + acc_ref[1]
        den = a[:, Hp:Hp + 1] + 1e-16
        xl = jax.nn.relu(a[:, :Hp] / den + b_ref[...])
        W = w_ref[...]
        s_col = W @ as_ref[...]
        d_col = W @ ad_ref[...]
        pad = ST - (H + 2)
        cols = [W, s_col[:, None], d_col[:, None]]
        if pad:
            cols.append(jnp.zeros((Hp, pad), jnp.float32))
        wbig = jnp.concatenate(cols, axis=1)
        t = jnp.dot(xl, wbig, preferred_element_type=jnp.float32)
        o_ref[...] = t
        d_ref[...] = t[:, H + 1]

    def run(acc, b, W, a_s, a_d):
        return pl.pallas_call(
            body,
            out_shape=(jax.ShapeDtypeStruct((N, ST), jnp.float32),
                       jax.ShapeDtypeStruct((N,), jnp.float32)),
        )(acc, b, W, a_s, a_d)

    return run


def _tc_prep4_body(acc_ref, b_ref, we_ref, p_ref, q_ref):
    a = acc_ref[0] + acc_ref[1]
    den = a[:, H3:H3 + 1] + 1e-16
    x3 = jax.nn.relu(a[:, :H3] / den + b_ref[...])
    zpad = jnp.zeros((N, 2), jnp.float32)
    p = jnp.dot(x3, we_ref[:H3, :], preferred_element_type=jnp.float32)
    q = jnp.dot(x3, we_ref[H3:2 * H3, :], preferred_element_type=jnp.float32)
    p_ref[...] = jnp.concatenate([p, zpad], axis=1)
    q_ref[...] = jnp.concatenate([q, zpad], axis=1)


def tc_prep4(acc, b3, We):
    return pl.pallas_call(
        _tc_prep4_body,
        out_shape=(jax.ShapeDtypeStruct((N, 12), jnp.float32),
                   jax.ShapeDtypeStruct((N, 12), jnp.float32)),
    )(acc, b3, We)


BEP = 16000


def _tc_eproj_body(e_ref, we_ref, be_ref, o_ref):
    r = lax.dot_general(we_ref[...], e_ref[...], (((0,), (1,)), ((), ())),
                        preferred_element_type=jnp.float32)
    r = r + be_ref[...][:, None]
    o_ref[...] = jnp.concatenate(
        [r, jnp.zeros((6, r.shape[1]), jnp.float32)], axis=0)


def tc_eproj(e, We, be):
    return pl.pallas_call(
        _tc_eproj_body,
        grid=(E // BEP,),
        in_specs=[
            pl.BlockSpec((BEP, D_E), lambda i: (i, 0)),
            pl.BlockSpec((D_E, 10), lambda i: (0, 0)),
            pl.BlockSpec((10,), lambda i: (0,)),
        ],
        out_specs=pl.BlockSpec((16, BEP), lambda i: (0, i)),
        out_shape=jax.ShapeDtypeStruct((16, E), jnp.float32),
    )(e, We[2 * H3:], be)


_sc_gat1 = _make_sc_gat(8, H1)
_sc_gat23 = _make_sc_gat(12, H2)
_sc_mlp = _make_sc_edge_mlp()
_tc_prep2 = _make_tc_prep_mid(H1, H2, 12)
_tc_prep3 = _make_tc_prep_mid(H2, H3, 12)


def kernel(x, e, edge_index, W1, a1s, a1d, b1, W2, a2s, a2d, b2,
           W3, a3s, a3d, b3, We, be, W9, b9):
    src = edge_index[0]
    dst = edge_index[1]
    wl = jnp.concatenate(
        [W9.reshape(-1), b9, jnp.zeros((4,), jnp.float32)])

    rt = tc_eproj(e, We, be)
    t1, d1 = tc_prep1(x, W1, a1s, a1d)
    acc1 = _sc_gat1(t1, d1, src, dst)
    t2, d2 = _tc_prep2(acc1, b1, W2, a2s, a2d)
    acc2 = _sc_gat23(t2, d2, src, dst)
    t3, d3 = _tc_prep3(acc2, b2, W3, a3s, a3d)
    acc3 = _sc_gat23(t3, d3, src, dst)
    tp, tq = tc_prep4(acc3, b3, We)
    return _sc_mlp(tp, tq, rt, wl, src, dst)


# speedup vs baseline: 1.0950x; 1.0073x over previous
import functools

import jax
import jax.numpy as jnp
from jax import lax
from jax.experimental import pallas as pl
from jax.experimental.pallas import tpu as pltpu
from jax.experimental.pallas import tpu_sc as plsc

N = 10000
E = 320000
D_IN = 128
D_E = 16
N_CLASSES = 4
H1, H2, H3 = 5, 10, 10

NC = 2
NS = 16
NW = NC * NS
C = 1280
NCHUNK = E // C
RPT = 624

_mesh = plsc.VectorSubcoreMesh(core_axis_name="c", subcore_axis_name="s")
_sc_params = pltpu.CompilerParams(needs_layout_passes=False,
                                  use_tc_tiling_on_sc=False)


def _iota16():
    return lax.iota(jnp.int32, 16)


def _full16(v):
    return jnp.full((16,), v, jnp.int32)


def _nk(wid):
    q, r = NCHUNK // NW, NCHUNK % NW
    if r == 0:
        return q
    return jnp.where(wid < r, q + 1, q)


def _make_sc_gat(ST, H):
    CG = 400
    NKG = 25

    @functools.partial(
        pl.kernel,
        out_type=jax.ShapeDtypeStruct((NC, N, ST), jnp.float32),
        mesh=_mesh,
        compiler_params=_sc_params,
        scratch_types=[
            pltpu.VMEM((N,), jnp.float32),
            pltpu.VMEM((CG,), jnp.int32),
            pltpu.VMEM((CG,), jnp.int32),
            pltpu.VMEM((CG, ST), jnp.float32),
            pltpu.VMEM((CG,), jnp.int32),
            pltpu.VMEM((CG,), jnp.int32),
            pltpu.VMEM((CG, ST), jnp.float32),
            pltpu.VMEM_SHARED((N, ST), jnp.float32),
            pltpu.SemaphoreType.DMA,
            pltpu.SemaphoreType.DMA,
            pltpu.SemaphoreType.DMA,
            pltpu.SemaphoreType.DMA,
        ],
    )
    def gat(table_hbm, d_hbm, src_hbm, dst_hbm, out_hbm,
            d_v, src_v0, dst_v0, rows_v0, src_v1, dst_v1, rows_v1,
            acc_sh, sem0, sem1, ssem0, ssem1):
        cid = lax.axis_index("c")
        sid = lax.axis_index("s")
        wid = sid * NC + cid
        ebase = wid * (NKG * CG)
        bufs = ((src_v0, dst_v0, rows_v0, sem0, ssem0),
                (src_v1, dst_v1, rows_v1, sem1, ssem1))

        z = jnp.zeros((16,), jnp.float32)
        ziota = _iota16()

        def zrow(t, carry):
            p = t * 16 + ziota
            plsc.store_scatter(rows_v0, [p // ST, p % ST], z)
            return carry

        lax.fori_loop(0, CG * ST // 16, zrow, 0)
        a0 = sid * RPT
        pltpu.sync_copy(rows_v0.at[pl.ds(0, CG)], acc_sh.at[pl.ds(a0, CG)])
        pltpu.sync_copy(rows_v0.at[pl.ds(0, RPT - CG)],
                        acc_sh.at[pl.ds(a0 + CG, RPT - CG)])

        @pl.when(sid == NS - 1)
        def _ztail():
            pltpu.sync_copy(rows_v0.at[pl.ds(0, 16)],
                            acc_sh.at[pl.ds(N - 16, 16)])

        pltpu.sync_copy(d_hbm, d_v)
        plsc.subcore_barrier()

        iota = _iota16()

        def start_gather(k, b):
            src_v, dst_v, rows_v, sem, _ = bufs[b]
            base = ebase + k * CG
            pltpu.sync_copy(src_hbm.at[pl.ds(base, CG)], src_v)
            pltpu.sync_copy(dst_hbm.at[pl.ds(base, CG)], dst_v)
            pltpu.async_copy(table_hbm.at[src_v], rows_v, sem)

        def wait_scatter(b):
            src_v, dst_v, rows_v, _, ssem = bufs[b]
            pltpu.make_async_copy(rows_v, acc_sh.at[dst_v], ssem).wait()

        def compute_chunk(b):
            src_v, dst_v, rows_v, sem, ssem = bufs[b]
            pltpu.make_async_copy(table_hbm.at[src_v], rows_v, sem).wait()
            for i in range(CG // 16):
                off = i * 16
                jrow = iota + off
                dj = dst_v[pl.ds(off, 16)]
                sv = plsc.load_gather(rows_v, [jrow, _full16(H)])
                dv = plsc.load_gather(d_v, [dj])
                l = sv + dv
                l = jnp.where(l >= 0.0, l, 0.2 * l)
                w = jnp.exp(l)
                for f in range(H):
                    cf = plsc.load_gather(rows_v, [jrow, _full16(f)])
                    plsc.store_scatter(rows_v, [jrow, _full16(f)], cf * w)
                plsc.store_scatter(rows_v, [jrow, _full16(H)], w)
                plsc.store_scatter(rows_v, [jrow, _full16(H + 1)], w)
            pltpu.make_async_copy(rows_v, acc_sh.at[dst_v], ssem).start(add=True)

        start_gather(0, 0)
        start_gather(1, 1)

        def pair_body(k2, carry):
            k = 2 * k2
            compute_chunk(0)
            compute_chunk(1)
            wait_scatter(0)
            start_gather(k + 2, 0)

            @pl.when(k2 < (NKG - 1) // 2 - 1)
            def _g1():
                wait_scatter(1)
                start_gather(k + 3, 1)

            return carry

        lax.fori_loop(0, (NKG - 1) // 2, pair_body, 0)
        compute_chunk(0)
        wait_scatter(1)
        wait_scatter(0)

        plsc.subcore_barrier()
        pltpu.sync_copy(acc_sh.at[pl.ds(a0, RPT)],
                        out_hbm.at[cid, pl.ds(a0, RPT)])

        @pl.when(sid == NS - 1)
        def _tail():
            pltpu.sync_copy(acc_sh.at[pl.ds(N - 16, 16)],
                            out_hbm.at[cid, pl.ds(N - 16, 16)])

    return gat


def _make_sc_edge_mlp():
    @functools.partial(
        pl.kernel,
        out_type=jax.ShapeDtypeStruct((E, N_CLASSES), jnp.float32),
        mesh=_mesh,
        compiler_params=_sc_params,
        scratch_types=[
            pltpu.VMEM((48,), jnp.float32),
            pltpu.VMEM((C,), jnp.int32),
            pltpu.VMEM((C,), jnp.int32),
            pltpu.VMEM((C, 12), jnp.float32),
            pltpu.VMEM((C, 12), jnp.float32),
            pltpu.VMEM((16, C), jnp.float32),
            pltpu.VMEM((C, N_CLASSES), jnp.float32),
            pltpu.SemaphoreType.DMA,
            pltpu.SemaphoreType.DMA,
        ],
    )
    def edge_mlp(tabp_hbm, tabq_hbm, rt_hbm, wl_hbm, src_hbm, dst_hbm,
                 out_hbm, wl_v, src_v, dst_v, rp_v, rq_v, rt_v, ob_v,
                 semp, semq):
        cid = lax.axis_index("c")
        sid = lax.axis_index("s")
        wid = sid * NC + cid

        pltpu.sync_copy(wl_hbm, wl_v)
        iota = _iota16()
        w9b = [[plsc.load_gather(wl_v, [_full16(j * 4 + c)])
                for c in range(N_CLASSES)] for j in range(10)]
        b9b = [plsc.load_gather(wl_v, [_full16(40 + c)])
               for c in range(N_CLASSES)]

        def chunk_body(k, carry):
            base = (wid + NW * k) * C
            pltpu.sync_copy(src_hbm.at[pl.ds(base, C)], src_v)
            pltpu.sync_copy(dst_hbm.at[pl.ds(base, C)], dst_v)
            cp = pltpu.async_copy(tabp_hbm.at[src_v], rp_v, semp)
            cq = pltpu.async_copy(tabq_hbm.at[dst_v], rq_v, semq)
            pltpu.sync_copy(rt_hbm.at[:, pl.ds(base, C)], rt_v)
            cp.wait()
            cq.wait()

            def group_body(i, gc):
                off = i * 16
                jrow = iota + off
                e2 = []
                for j in range(10):
                    a = (plsc.load_gather(rp_v, [jrow, _full16(j)])
                         + plsc.load_gather(rq_v, [jrow, _full16(j)])
                         + rt_v[j, pl.ds(off, 16)])
                    e2.append(jnp.maximum(a, 0.0))
                oc = []
                for c in range(N_CLASSES):
                    o = b9b[c]
                    for j in range(10):
                        o = o + e2[j] * w9b[j][c]
                    oc.append(jnp.maximum(o, 0.0))
                m = jnp.maximum(jnp.maximum(oc[0], oc[1]),
                                jnp.maximum(oc[2], oc[3]))
                ex = [jnp.exp(o - m) for o in oc]
                s = ex[0] + ex[1] + ex[2] + ex[3]
                for c in range(N_CLASSES):
                    plsc.store_scatter(ob_v, [jrow, _full16(c)], ex[c] / s)
                return gc

            lax.fori_loop(0, C // 16, group_body, 0)
            pltpu.sync_copy(ob_v, out_hbm.at[pl.ds(base, C)])
            return carry

        lax.fori_loop(0, _nk(wid), chunk_body, 0)

    return edge_mlp


def _tc_prep1_body(x_ref, w_ref, as_ref, ad_ref, o_ref, d_ref):
    W = w_ref[...]
    s_col = W @ as_ref[...]
    d_col = W @ ad_ref[...]
    wbig = jnp.concatenate(
        [W, s_col[:, None], d_col[:, None], jnp.zeros((D_IN, 1), jnp.float32)],
        axis=1)
    t = jnp.dot(x_ref[...], wbig, preferred_element_type=jnp.float32)
    o_ref[...] = t
    d_ref[...] = t[:, H1 + 1]


def tc_prep1(x, W1, a1s, a1d):
    return pl.pallas_call(
        _tc_prep1_body,
        out_shape=(jax.ShapeDtypeStruct((N, 8), jnp.float32),
                   jax.ShapeDtypeStruct((N,), jnp.float32)),
    )(x, W1, a1s, a1d)


def _make_tc_prep_mid(Hp, H, ST):
    def body(acc_ref, b_ref, w_ref, as_ref, ad_ref, o_ref, d_ref):
        a = acc_ref[0] + acc_ref[1]
        den = a[:, Hp:Hp + 1] + 1e-16
        xl = jax.nn.relu(a[:, :Hp] / den + b_ref[...])
        W = w_ref[...]
        s_col = W @ as_ref[...]
        d_col = W @ ad_ref[...]
        pad = ST - (H + 2)
        cols = [W, s_col[:, None], d_col[:, None]]
        if pad:
            cols.append(jnp.zeros((Hp, pad), jnp.float32))
        wbig = jnp.concatenate(cols, axis=1)
        t = jnp.dot(xl, wbig, preferred_element_type=jnp.float32)
        o_ref[...] = t
        d_ref[...] = t[:, H + 1]

    def run(acc, b, W, a_s, a_d):
        return pl.pallas_call(
            body,
            out_shape=(jax.ShapeDtypeStruct((N, ST), jnp.float32),
                       jax.ShapeDtypeStruct((N,), jnp.float32)),
        )(acc, b, W, a_s, a_d)

    return run


def _tc_prep4_body(acc_ref, b_ref, we_ref, p_ref, q_ref):
    a = acc_ref[0] + acc_ref[1]
    den = a[:, H3:H3 + 1] + 1e-16
    x3 = jax.nn.relu(a[:, :H3] / den + b_ref[...])
    zpad = jnp.zeros((N, 2), jnp.float32)
    p = jnp.dot(x3, we_ref[:H3, :], preferred_element_type=jnp.float32)
    q = jnp.dot(x3, we_ref[H3:2 * H3, :], preferred_element_type=jnp.float32)
    p_ref[...] = jnp.concatenate([p, zpad], axis=1)
    q_ref[...] = jnp.concatenate([q, zpad], axis=1)


def tc_prep4(acc, b3, We):
    return pl.pallas_call(
        _tc_prep4_body,
        out_shape=(jax.ShapeDtypeStruct((N, 12), jnp.float32),
                   jax.ShapeDtypeStruct((N, 12), jnp.float32)),
    )(acc, b3, We)


BEP = 16000


def _tc_eproj_body(e_ref, we_ref, be_ref, o_ref):
    r = lax.dot_general(we_ref[...], e_ref[...], (((0,), (1,)), ((), ())),
                        preferred_element_type=jnp.float32)
    r = r + be_ref[...][:, None]
    o_ref[...] = jnp.concatenate(
        [r, jnp.zeros((6, r.shape[1]), jnp.float32)], axis=0)


def tc_eproj(e, We, be):
    return pl.pallas_call(
        _tc_eproj_body,
        grid=(E // BEP,),
        in_specs=[
            pl.BlockSpec((BEP, D_E), lambda i: (i, 0)),
            pl.BlockSpec((D_E, 10), lambda i: (0, 0)),
            pl.BlockSpec((10,), lambda i: (0,)),
        ],
        out_specs=pl.BlockSpec((16, BEP), lambda i: (0, i)),
        out_shape=jax.ShapeDtypeStruct((16, E), jnp.float32),
    )(e, We[2 * H3:], be)


_sc_gat1 = _make_sc_gat(8, H1)
_sc_gat23 = _make_sc_gat(12, H2)
_sc_mlp = _make_sc_edge_mlp()
_tc_prep2 = _make_tc_prep_mid(H1, H2, 12)
_tc_prep3 = _make_tc_prep_mid(H2, H3, 12)


def kernel(x, e, edge_index, W1, a1s, a1d, b1, W2, a2s, a2d, b2,
           W3, a3s, a3d, b3, We, be, W9, b9):
    src = edge_index[0]
    dst = edge_index[1]
    wl = jnp.concatenate(
        [W9.reshape(-1), b9, jnp.zeros((4,), jnp.float32)])

    rt = tc_eproj(e, We, be)
    t1, d1 = tc_prep1(x, W1, a1s, a1d)
    acc1 = _sc_gat1(t1, d1, src, dst)
    t2, d2 = _tc_prep2(acc1, b1, W2, a2s, a2d)
    acc2 = _sc_gat23(t2, d2, src, dst)
    t3, d3 = _tc_prep3(acc2, b2, W3, a3s, a3d)
    acc3 = _sc_gat23(t3, d3, src, dst)
    tp, tq = tc_prep4(acc3, b3, We)
    return _sc_mlp(tp, tq, rt, wl, src, dst)


# speedup vs baseline: 1.0963x; 1.0012x over previous
import functools

import jax
import jax.numpy as jnp
from jax import lax
from jax.experimental import pallas as pl
from jax.experimental.pallas import tpu as pltpu
from jax.experimental.pallas import tpu_sc as plsc

N = 10000
E = 320000
D_IN = 128
D_E = 16
N_CLASSES = 4
H1, H2, H3 = 5, 10, 10

NC = 2
NS = 16
NW = NC * NS
C = 1280
NCHUNK = E // C
RPT = 624

_mesh = plsc.VectorSubcoreMesh(core_axis_name="c", subcore_axis_name="s")
_sc_params = pltpu.CompilerParams(needs_layout_passes=False,
                                  use_tc_tiling_on_sc=False)


def _iota16():
    return lax.iota(jnp.int32, 16)


def _full16(v):
    return jnp.full((16,), v, jnp.int32)


def _nk(wid):
    q, r = NCHUNK // NW, NCHUNK % NW
    if r == 0:
        return q
    return jnp.where(wid < r, q + 1, q)


def _make_sc_gat(ST, H):
    CG = 400
    NKG = 25

    @functools.partial(
        pl.kernel,
        out_type=jax.ShapeDtypeStruct((NC, N, ST), jnp.float32),
        mesh=_mesh,
        compiler_params=_sc_params,
        scratch_types=[
            pltpu.VMEM((N,), jnp.float32),
            pltpu.VMEM((CG,), jnp.int32),
            pltpu.VMEM((CG,), jnp.int32),
            pltpu.VMEM((CG, ST), jnp.float32),
            pltpu.VMEM((CG,), jnp.int32),
            pltpu.VMEM((CG,), jnp.int32),
            pltpu.VMEM((CG, ST), jnp.float32),
            pltpu.VMEM_SHARED((N, ST), jnp.float32),
            pltpu.SemaphoreType.DMA,
            pltpu.SemaphoreType.DMA,
            pltpu.SemaphoreType.DMA,
            pltpu.SemaphoreType.DMA,
        ],
    )
    def gat(table_hbm, d_hbm, src_hbm, dst_hbm, out_hbm,
            d_v, src_v0, dst_v0, rows_v0, src_v1, dst_v1, rows_v1,
            acc_sh, sem0, sem1, ssem0, ssem1):
        cid = lax.axis_index("c")
        sid = lax.axis_index("s")
        wid = sid * NC + cid
        ebase = wid * (NKG * CG)
        bufs = ((src_v0, dst_v0, rows_v0, sem0, ssem0),
                (src_v1, dst_v1, rows_v1, sem1, ssem1))

        z = jnp.zeros((16,), jnp.float32)
        ziota = _iota16()

        def zrow(t, carry):
            p = t * 16 + ziota
            plsc.store_scatter(rows_v0, [p // ST, p % ST], z)
            return carry

        lax.fori_loop(0, CG * ST // 16, zrow, 0)
        a0 = sid * RPT
        pltpu.sync_copy(rows_v0.at[pl.ds(0, CG)], acc_sh.at[pl.ds(a0, CG)])
        pltpu.sync_copy(rows_v0.at[pl.ds(0, RPT - CG)],
                        acc_sh.at[pl.ds(a0 + CG, RPT - CG)])

        @pl.when(sid == NS - 1)
        def _ztail():
            pltpu.sync_copy(rows_v0.at[pl.ds(0, 16)],
                            acc_sh.at[pl.ds(N - 16, 16)])

        pltpu.sync_copy(d_hbm, d_v)
        plsc.subcore_barrier()

        iota = _iota16()

        def start_gather(k, b):
            src_v, dst_v, rows_v, sem, _ = bufs[b]
            base = ebase + k * CG
            pltpu.sync_copy(src_hbm.at[pl.ds(base, CG)], src_v)
            pltpu.sync_copy(dst_hbm.at[pl.ds(base, CG)], dst_v)
            pltpu.async_copy(table_hbm.at[src_v], rows_v, sem)

        def wait_scatter(b):
            src_v, dst_v, rows_v, _, ssem = bufs[b]
            pltpu.make_async_copy(rows_v, acc_sh.at[dst_v], ssem).wait()

        def compute_chunk(b):
            src_v, dst_v, rows_v, sem, ssem = bufs[b]
            pltpu.make_async_copy(table_hbm.at[src_v], rows_v, sem).wait()
            for i in range(CG // 16):
                off = i * 16
                jrow = iota + off
                dj = dst_v[pl.ds(off, 16)]
                sv = plsc.load_gather(rows_v, [jrow, _full16(H)])
                dv = plsc.load_gather(d_v, [dj])
                l = sv + dv
                l = jnp.where(l >= 0.0, l, 0.2 * l)
                w = jnp.exp(l)
                for f in range(H):
                    cf = plsc.load_gather(rows_v, [jrow, _full16(f)])
                    plsc.store_scatter(rows_v, [jrow, _full16(f)], cf * w)
                plsc.store_scatter(rows_v, [jrow, _full16(H)], w)
                plsc.store_scatter(rows_v, [jrow, _full16(H + 1)], w)
            pltpu.make_async_copy(rows_v, acc_sh.at[dst_v], ssem).start(add=True)

        start_gather(0, 0)
        start_gather(1, 1)

        def pair_body(k2, carry):
            k = 2 * k2
            compute_chunk(0)
            compute_chunk(1)
            wait_scatter(0)
            start_gather(k + 2, 0)

            @pl.when(k2 < (NKG - 1) // 2 - 1)
            def _g1():
                wait_scatter(1)
                start_gather(k + 3, 1)

            return carry

        lax.fori_loop(0, (NKG - 1) // 2, pair_body, 0)
        compute_chunk(0)
        wait_scatter(1)
        wait_scatter(0)

        plsc.subcore_barrier()
        pltpu.sync_copy(acc_sh.at[pl.ds(a0, RPT)],
                        out_hbm.at[cid, pl.ds(a0, RPT)])

        @pl.when(sid == NS - 1)
        def _tail():
            pltpu.sync_copy(acc_sh.at[pl.ds(N - 16, 16)],
                            out_hbm.at[cid, pl.ds(N - 16, 16)])

    return gat


def _make_sc_edge_mlp():
    @functools.partial(
        pl.kernel,
        out_type=jax.ShapeDtypeStruct((E, N_CLASSES), jnp.float32),
        mesh=_mesh,
        compiler_params=_sc_params,
        scratch_types=[
            pltpu.VMEM((48,), jnp.float32),
            pltpu.VMEM((C,), jnp.int32),
            pltpu.VMEM((C,), jnp.int32),
            pltpu.VMEM((C, 10), jnp.float32),
            pltpu.VMEM((C, 10), jnp.float32),
            pltpu.VMEM((10, C), jnp.float32),
            pltpu.VMEM((C, N_CLASSES), jnp.float32),
            pltpu.SemaphoreType.DMA,
            pltpu.SemaphoreType.DMA,
        ],
    )
    def edge_mlp(tabp_hbm, tabq_hbm, rt_hbm, wl_hbm, src_hbm, dst_hbm,
                 out_hbm, wl_v, src_v, dst_v, rp_v, rq_v, rt_v, ob_v,
                 semp, semq):
        cid = lax.axis_index("c")
        sid = lax.axis_index("s")
        wid = sid * NC + cid

        pltpu.sync_copy(wl_hbm, wl_v)
        iota = _iota16()
        w9b = [[plsc.load_gather(wl_v, [_full16(j * 4 + c)])
                for c in range(N_CLASSES)] for j in range(10)]
        b9b = [plsc.load_gather(wl_v, [_full16(40 + c)])
               for c in range(N_CLASSES)]

        def chunk_body(k, carry):
            base = (wid + NW * k) * C
            pltpu.sync_copy(src_hbm.at[pl.ds(base, C)], src_v)
            pltpu.sync_copy(dst_hbm.at[pl.ds(base, C)], dst_v)
            cp = pltpu.async_copy(tabp_hbm.at[src_v], rp_v, semp)
            cq = pltpu.async_copy(tabq_hbm.at[dst_v], rq_v, semq)
            pltpu.sync_copy(rt_hbm.at[pl.ds(0, 10), pl.ds(base, C)], rt_v)
            cp.wait()
            cq.wait()

            def group_body(i, gc):
                off = i * 16
                jrow = iota + off
                e2 = []
                for j in range(10):
                    a = (plsc.load_gather(rp_v, [jrow, _full16(j)])
                         + plsc.load_gather(rq_v, [jrow, _full16(j)])
                         + rt_v[j, pl.ds(off, 16)])
                    e2.append(jnp.maximum(a, 0.0))
                oc = []
                for c in range(N_CLASSES):
                    o = b9b[c]
                    for j in range(10):
                        o = o + e2[j] * w9b[j][c]
                    oc.append(jnp.maximum(o, 0.0))
                m = jnp.maximum(jnp.maximum(oc[0], oc[1]),
                                jnp.maximum(oc[2], oc[3]))
                ex = [jnp.exp(o - m) for o in oc]
                s = ex[0] + ex[1] + ex[2] + ex[3]
                for c in range(N_CLASSES):
                    plsc.store_scatter(ob_v, [jrow, _full16(c)], ex[c] / s)
                return gc

            lax.fori_loop(0, C // 16, group_body, 0)
            pltpu.sync_copy(ob_v, out_hbm.at[pl.ds(base, C)])
            return carry

        lax.fori_loop(0, _nk(wid), chunk_body, 0)

    return edge_mlp


def _tc_prep1_body(x_ref, w_ref, as_ref, ad_ref, o_ref, d_ref):
    W = w_ref[...]
    s_col = W @ as_ref[...]
    d_col = W @ ad_ref[...]
    wbig = jnp.concatenate(
        [W, s_col[:, None], d_col[:, None], jnp.zeros((D_IN, 1), jnp.float32)],
        axis=1)
    t = jnp.dot(x_ref[...], wbig, preferred_element_type=jnp.float32)
    o_ref[...] = t
    d_ref[...] = t[:, H1 + 1]


def tc_prep1(x, W1, a1s, a1d):
    return pl.pallas_call(
        _tc_prep1_body,
        out_shape=(jax.ShapeDtypeStruct((N, 8), jnp.float32),
                   jax.ShapeDtypeStruct((N,), jnp.float32)),
    )(x, W1, a1s, a1d)


def _make_tc_prep_mid(Hp, H, ST):
    def body(acc_ref, b_ref, w_ref, as_ref, ad_ref, o_ref, d_ref):
        a = acc_ref[0] + acc_ref[1]
        den = a[:, Hp:Hp + 1] + 1e-16
        xl = jax.nn.relu(a[:, :Hp] / den + b_ref[...])
        W = w_ref[...]
        s_col = W @ as_ref[...]
        d_col = W @ ad_ref[...]
        pad = ST - (H + 2)
        cols = [W, s_col[:, None], d_col[:, None]]
        if pad:
            cols.append(jnp.zeros((Hp, pad), jnp.float32))
        wbig = jnp.concatenate(cols, axis=1)
        t = jnp.dot(xl, wbig, preferred_element_type=jnp.float32)
        o_ref[...] = t
        d_ref[...] = t[:, H + 1]

    def run(acc, b, W, a_s, a_d):
        return pl.pallas_call(
            body,
            out_shape=(jax.ShapeDtypeStruct((N, ST), jnp.float32),
                       jax.ShapeDtypeStruct((N,), jnp.float32)),
        )(acc, b, W, a_s, a_d)

    return run


def _tc_prep4_body(acc_ref, b_ref, we_ref, p_ref, q_ref):
    a = acc_ref[0] + acc_ref[1]
    den = a[:, H3:H3 + 1] + 1e-16
    x3 = jax.nn.relu(a[:, :H3] / den + b_ref[...])
    p_ref[...] = jnp.dot(x3, we_ref[:H3, :], preferred_element_type=jnp.float32)
    q_ref[...] = jnp.dot(x3, we_ref[H3:2 * H3, :],
                         preferred_element_type=jnp.float32)


def tc_prep4(acc, b3, We):
    return pl.pallas_call(
        _tc_prep4_body,
        out_shape=(jax.ShapeDtypeStruct((N, 10), jnp.float32),
                   jax.ShapeDtypeStruct((N, 10), jnp.float32)),
    )(acc, b3, We)


BEP = 16000


def _tc_eproj_body(e_ref, we_ref, be_ref, o_ref):
    r = lax.dot_general(we_ref[...], e_ref[...], (((0,), (1,)), ((), ())),
                        preferred_element_type=jnp.float32)
    r = r + be_ref[...][:, None]
    o_ref[...] = jnp.concatenate(
        [r, jnp.zeros((6, r.shape[1]), jnp.float32)], axis=0)


def tc_eproj(e, We, be):
    return pl.pallas_call(
        _tc_eproj_body,
        grid=(E // BEP,),
        in_specs=[
            pl.BlockSpec((BEP, D_E), lambda i: (i, 0)),
            pl.BlockSpec((D_E, 10), lambda i: (0, 0)),
            pl.BlockSpec((10,), lambda i: (0,)),
        ],
        out_specs=pl.BlockSpec((16, BEP), lambda i: (0, i)),
        out_shape=jax.ShapeDtypeStruct((16, E), jnp.float32),
    )(e, We[2 * H3:], be)


_sc_gat1 = _make_sc_gat(8, H1)
_sc_gat23 = _make_sc_gat(12, H2)
_sc_mlp = _make_sc_edge_mlp()
_tc_prep2 = _make_tc_prep_mid(H1, H2, 12)
_tc_prep3 = _make_tc_prep_mid(H2, H3, 12)


def kernel(x, e, edge_index, W1, a1s, a1d, b1, W2, a2s, a2d, b2,
           W3, a3s, a3d, b3, We, be, W9, b9):
    src = edge_index[0]
    dst = edge_index[1]
    wl = jnp.concatenate(
        [W9.reshape(-1), b9, jnp.zeros((4,), jnp.float32)])

    rt = tc_eproj(e, We, be)
    t1, d1 = tc_prep1(x, W1, a1s, a1d)
    acc1 = _sc_gat1(t1, d1, src, dst)
    t2, d2 = _tc_prep2(acc1, b1, W2, a2s, a2d)
    acc2 = _sc_gat23(t2, d2, src, dst)
    t3, d3 = _tc_prep3(acc2, b2, W3, a3s, a3d)
    acc3 = _sc_gat23(t3, d3, src, dst)
    tp, tq = tc_prep4(acc3, b3, We)
    return _sc_mlp(tp, tq, rt, wl, src, dst)
